# Initial kernel scaffold; baseline (speedup 1.0000x reference)
import jax
import jax.numpy as jnp
from jax.experimental import pallas as pl


def kernel(x, edge_index, W1, b1, W2, b2, Wfc, bfc):
    raise NotImplementedError("write your pallas kernel here")


# speedup vs baseline: 3.5172x; 3.5172x over previous
import functools

import jax
import jax.numpy as jnp
from jax import lax
from jax.experimental import pallas as pl
from jax.experimental.pallas import tpu as pltpu
from jax.experimental.pallas import tpu_sc as plsc

N = 1024
E = 16384
F = 50176
H1 = 1024
H2 = 256

NC, NS, L = 2, 16, 16
NW = NC * NS
EPT = E // NW
SLICE = (N * N) // NS
ZCH = 8192


def _sc_count_body(edges_hbm, out_hbm, ev, idx4, ones_v, zeros_v, cshared):
    cid = lax.axis_index("c")
    sid = lax.axis_index("s")

    def _zfill(i, _):
        zeros_v[pl.ds(i * L, L)] = jnp.zeros((L,), jnp.float32)
        return 0
    lax.fori_loop(0, ZCH // L, _zfill, 0)
    for r in range(128 // L):
        ones_v[pl.ds(r * L, L)] = jnp.full((L,), 1.0, jnp.float32)
    for j in range(SLICE // ZCH):
        pltpu.sync_copy(zeros_v, cshared.at[pl.ds(sid * SLICE + j * ZCH, ZCH)])
    plsc.subcore_barrier()

    base = (cid * NS + sid) * EPT
    pltpu.sync_copy(edges_hbm.at[:, pl.ds(base, EPT)], ev)
    for c in range(EPT // L):
        s = ev[0, pl.ds(c * L, L)]
        d = ev[1, pl.ds(c * L, L)]
        idx4[c // 8, pl.ds((c % 8) * L, L)] = d * N + s

    for r in range(EPT // 128):
        pltpu.sync_copy(ones_v, cshared.at[idx4.at[r]], add=True)
    plsc.subcore_barrier()

    pltpu.sync_copy(cshared.at[pl.ds(sid * SLICE, SLICE)],
                    out_hbm.at[cid, pl.ds(sid * SLICE, SLICE)])


_sc_count = functools.partial(
    pl.kernel,
    out_type=jax.ShapeDtypeStruct((NC, N * N), jnp.float32),
    mesh=plsc.VectorSubcoreMesh(core_axis_name="c", subcore_axis_name="s"),
    scratch_types=[
        pltpu.VMEM((2, EPT), jnp.int32),
        pltpu.VMEM((EPT // 128, 128), jnp.int32),
        pltpu.VMEM((128,), jnp.float32),
        pltpu.VMEM((ZCH,), jnp.float32),
        pltpu.VMEM_SHARED((N * N,), jnp.float32),
    ],
)(_sc_count_body)


KB = 512
KSTEPS = F // KB


def _mm_body(x_ref, w_ref, o_ref, acc_ref):
    k = pl.program_id(0)

    @pl.when(k == 0)
    def _init():
        acc_ref[...] = jnp.zeros_like(acc_ref)

    acc_ref[...] += jnp.dot(x_ref[...], w_ref[...],
                            preferred_element_type=jnp.float32)

    @pl.when(k == KSTEPS - 1)
    def _done():
        o_ref[...] = acc_ref[...]


def _mm(x, w1):
    return pl.pallas_call(
        _mm_body,
        grid=(KSTEPS,),
        in_specs=[
            pl.BlockSpec((N, KB), lambda k: (0, k)),
            pl.BlockSpec((KB, H1), lambda k: (k, 0)),
        ],
        out_specs=pl.BlockSpec((N, H1), lambda k: (0, 0)),
        out_shape=jax.ShapeDtypeStruct((N, H1), jnp.float32),
        scratch_shapes=[pltpu.VMEM((N, H1), jnp.float32)],
    )(x, w1)


def _epi_body(cp_ref, h_ref, b1_ref, w2_ref, b2_ref, wfc_ref, bfc_ref, o_ref):
    c = cp_ref[0] + cp_ref[1]
    rows = lax.broadcasted_iota(jnp.int32, (N, N), 0)
    cols = lax.broadcasted_iota(jnp.int32, (N, N), 1)
    cp = c + jnp.where(rows == cols, 1.0, 0.0)
    deg = jnp.sum(cp, axis=1, keepdims=True)
    dinv = lax.rsqrt(jnp.maximum(deg, 1.0))

    h1 = jnp.dot(cp, dinv * h_ref[...], preferred_element_type=jnp.float32)
    h1 = jnp.maximum(dinv * h1 + b1_ref[...], 0.0)
    t = jnp.dot(h1, w2_ref[...], preferred_element_type=jnp.float32)
    h2 = jnp.dot(cp, dinv * t, preferred_element_type=jnp.float32)
    h2 = jnp.maximum(dinv * h2 + b2_ref[...], 0.0)

    pooled = jnp.mean(h2, axis=0, keepdims=True)
    logits = jnp.dot(pooled, wfc_ref[...],
                     preferred_element_type=jnp.float32) + bfc_ref[...]
    m = jnp.max(logits, axis=1, keepdims=True)
    ex = jnp.exp(logits - m)
    o_ref[...] = (logits - m) - jnp.log(jnp.sum(ex, axis=1, keepdims=True))


def _epilogue(cpart, h, b1, w2, b2, wfc, bfc):
    return pl.pallas_call(
        _epi_body,
        out_shape=jax.ShapeDtypeStruct((1, 2), jnp.float32),
    )(cpart, h, b1, w2, b2, wfc, bfc)


def kernel(x, edge_index, W1, b1, W2, b2, Wfc, bfc):
    ei = edge_index.astype(jnp.int32)
    cpart = _sc_count(ei).reshape(NC, N, N)
    h = _mm(x.astype(jnp.float32), W1)
    return _epilogue(cpart, h, b1.reshape(1, H1), W2, b2.reshape(1, H2),
                     Wfc, bfc.reshape(1, 2))
